# baseline (speedup 1.0000x reference)
import functools
import jax
import jax.numpy as jnp
from jax.experimental import pallas as pl
from jax.experimental.pallas import tpu as pltpu

_BN_EPS = 1e-5


def _conv9_body(xp_ref, w_ref, s_ref, t_ref, o_ref, *, dil, rows, wimg, act):
    cin = xp_ref.shape[3]
    cout = o_ref.shape[2]
    himg = o_ref.shape[1] // wimg
    for r0 in range(0, himg, rows):
        acc = jnp.zeros((rows * wimg, cout), jnp.float32)
        for ky in range(3):
            for kx in range(3):
                k = ky * 3 + kx
                xs = xp_ref[0, r0 + ky * dil:r0 + ky * dil + rows,
                            kx * dil:kx * dil + wimg, :]
                acc = acc + jnp.dot(xs.reshape(rows * wimg, cin), w_ref[k],
                                    preferred_element_type=jnp.float32)
        y = acc * s_ref[...] + t_ref[...]
        if act == 'relu':
            y = jnp.maximum(y, 0.0)
        elif act == 'sig':
            y = jnp.clip(jax.nn.sigmoid(y), 1e-4, 1.0 - 1e-4)
        o_ref[0, r0 * wimg:(r0 + rows) * wimg, :] = y


def _conv9(xp, w9, s, t, act, dil, rows=32):
    b, hp, wp, cin = xp.shape
    cout = w9.shape[2]
    himg, wimg = hp - 2 * dil, wp - 2 * dil
    body = functools.partial(_conv9_body, dil=dil, rows=rows, wimg=wimg, act=act)
    return pl.pallas_call(
        body,
        grid=(b,),
        in_specs=[
            pl.BlockSpec((1, hp, wp, cin), lambda i: (i, 0, 0, 0)),
            pl.BlockSpec((9, cin, cout), lambda i: (0, 0, 0)),
            pl.BlockSpec((1, cout), lambda i: (0, 0)),
            pl.BlockSpec((1, cout), lambda i: (0, 0)),
        ],
        out_specs=pl.BlockSpec((1, himg * wimg, cout), lambda i: (i, 0, 0)),
        out_shape=jax.ShapeDtypeStruct((b, himg * wimg, cout), jnp.float32),
        compiler_params=pltpu.CompilerParams(dimension_semantics=("parallel",)),
    )(xp, w9, s, t)


def _dcn_body(g_ref, w_ref, s_ref, t_ref, o_ref):
    blk, c = g_ref.shape[2], g_ref.shape[3]
    acc = jnp.zeros((blk, w_ref.shape[2]), jnp.float32)
    for k in range(9):
        acc = acc + jnp.dot(g_ref[0, k], w_ref[k],
                            preferred_element_type=jnp.float32)
    o_ref[0] = jnp.maximum(acc * s_ref[...] + t_ref[...], 0.0)


def _dcn_contract(g, w9, s, t, blk=4096):
    b, _, hw, c = g.shape
    return pl.pallas_call(
        _dcn_body,
        grid=(b, hw // blk),
        in_specs=[
            pl.BlockSpec((1, 9, blk, c), lambda i, j: (i, 0, j, 0)),
            pl.BlockSpec((9, c, c), lambda i, j: (0, 0, 0)),
            pl.BlockSpec((1, c), lambda i, j: (0, 0)),
            pl.BlockSpec((1, c), lambda i, j: (0, 0)),
        ],
        out_specs=pl.BlockSpec((1, blk, c), lambda i, j: (i, j, 0)),
        out_shape=jax.ShapeDtypeStruct((b, hw, c), jnp.float32),
        compiler_params=pltpu.CompilerParams(
            dimension_semantics=("parallel", "arbitrary")),
    )(g, w9, s, t)


def _conv9_split(xp, w9, s, t, act, dil):
    himg = xp.shape[1] - 2 * dil
    part = himg // 4
    outs = [_conv9(xp[:, p * part:(p + 1) * part + 2 * dil], w9, s, t, act, dil)
            for p in range(4)]
    return jnp.concatenate(outs, axis=1)


def _tap_w(w):
    return w.transpose(2, 3, 1, 0).reshape(9, w.shape[1], w.shape[0])


def kernel(x, w_om, b_om, w_dcn, b_dcn,
           bn1_g, bn1_b, bn1_m, bn1_v,
           w_h, b_h, bn2_g, bn2_b, bn2_m, bn2_v,
           w_w, b_w, bn3_g, bn3_b, bn3_m, bn3_v,
           w3, b3):
    b, c, h, w = x.shape
    hw = h * w
    f32 = jnp.float32

    xh = x.transpose(0, 2, 3, 1)
    xp1 = jnp.pad(xh, ((0, 0), (1, 1), (1, 1), (0, 0)))

    om = _conv9_split(xp1, _tap_w(w_om), jnp.ones((1, 27), f32),
                b_om.reshape(1, 27), act=None, dil=1)

    o1 = om[:, :, 0:9].transpose(0, 2, 1)
    o2 = om[:, :, 9:18].transpose(0, 2, 1)
    mk = jax.nn.sigmoid(om[:, :, 18:27]).transpose(0, 2, 1)

    kyv = jnp.repeat(jnp.arange(3, dtype=f32) - 1, 3)
    kxv = jnp.tile(jnp.arange(3, dtype=f32) - 1, 3)
    rows = (jnp.arange(hw, dtype=jnp.int32) // w).astype(f32)
    cols = (jnp.arange(hw, dtype=jnp.int32) % w).astype(f32)
    py = rows[None, None, :] + kyv[None, :, None] + o1
    px = cols[None, None, :] + kxv[None, :, None] + o2
    y0 = jnp.floor(py)
    x0 = jnp.floor(px)
    wy = py - y0
    wx = px - x0
    y0i = y0.astype(jnp.int32)
    x0i = x0.astype(jnp.int32)

    x_hwc = xh.reshape(b, hw, c)
    g = jnp.zeros((b, 9 * hw, c), f32)
    for dy in (0, 1):
        for dx in (0, 1):
            yi = y0i + dy
            xi = x0i + dx
            valid = (yi >= 0) & (yi < h) & (xi >= 0) & (xi < w)
            wc = jnp.where(valid,
                           (wy if dy else 1.0 - wy) * (wx if dx else 1.0 - wx),
                           0.0) * mk
            idx = jnp.clip(yi, 0, h - 1) * w + jnp.clip(xi, 0, w - 1)
            gath = jax.vmap(lambda xf, i: xf[i])(x_hwc, idx.reshape(b, -1))
            g = g + gath * wc.reshape(b, -1)[..., None]
    g = g.reshape(b, 9, hw, c)

    s1 = bn1_g * jax.lax.rsqrt(bn1_v + _BN_EPS)
    t1 = (b_dcn - bn1_m) * s1 + bn1_b
    h1 = _dcn_contract(g, _tap_w(w_dcn), s1.reshape(1, c), t1.reshape(1, c))

    s2 = bn2_g * jax.lax.rsqrt(bn2_v + _BN_EPS)
    t2 = (b_h - bn2_m) * s2 + bn2_b
    hp = jnp.pad(h1.reshape(b, h, w, c), ((0, 0), (2, 2), (2, 2), (0, 0)))
    h2 = _conv9_split(hp, _tap_w(w_h), s2.reshape(1, c), t2.reshape(1, c),
                act='relu', dil=2)

    s3 = bn3_g * jax.lax.rsqrt(bn3_v + _BN_EPS)
    t3 = (b_w - bn3_m) * s3 + bn3_b
    hp = jnp.pad(h2.reshape(b, h, w, c), ((0, 0), (2, 2), (2, 2), (0, 0)))
    h3 = _conv9_split(hp, _tap_w(w_w), s3.reshape(1, c), t3.reshape(1, c),
                act='relu', dil=2)

    hp = jnp.pad(h3.reshape(b, h, w, c), ((0, 0), (1, 1), (1, 1), (0, 0)))
    out = _conv9_split(hp, _tap_w(w3), jnp.ones((1, 1), f32), b3.reshape(1, 1),
                 act='sig', dil=1)
    return out.reshape(b, h, w, 1).transpose(0, 3, 1, 2)


# speedup vs baseline: 2.4695x; 2.4695x over previous
import functools
import jax
import jax.numpy as jnp
from jax.experimental import pallas as pl
from jax.experimental.pallas import tpu as pltpu

_BN_EPS = 1e-5


def _conv9_body(xp_ref, w_ref, s_ref, t_ref, o_ref, *, dil, rows, wimg, act):
    cin = xp_ref.shape[3]
    cout = o_ref.shape[2]
    himg = o_ref.shape[1] // wimg
    for r0 in range(0, himg, rows):
        acc = jnp.zeros((rows * wimg, cout), jnp.float32)
        for ky in range(3):
            for kx in range(3):
                k = ky * 3 + kx
                xs = xp_ref[0, r0 + ky * dil:r0 + ky * dil + rows,
                            kx * dil:kx * dil + wimg, :]
                acc = acc + jnp.dot(xs.reshape(rows * wimg, cin), w_ref[k],
                                    preferred_element_type=jnp.float32)
        y = acc * s_ref[...] + t_ref[...]
        if act == 'relu':
            y = jnp.maximum(y, 0.0)
        elif act == 'sig':
            y = jnp.clip(jax.nn.sigmoid(y), 1e-4, 1.0 - 1e-4)
        o_ref[0, r0 * wimg:(r0 + rows) * wimg, :] = y


def _conv9(xp, w9, s, t, act, dil, rows=32):
    b, hp, wp, cin = xp.shape
    cout = w9.shape[2]
    himg, wimg = hp - 2 * dil, wp - 2 * dil
    body = functools.partial(_conv9_body, dil=dil, rows=rows, wimg=wimg, act=act)
    return pl.pallas_call(
        body,
        grid=(b,),
        in_specs=[
            pl.BlockSpec((1, hp, wp, cin), lambda i: (i, 0, 0, 0)),
            pl.BlockSpec((9, cin, cout), lambda i: (0, 0, 0)),
            pl.BlockSpec((1, cout), lambda i: (0, 0)),
            pl.BlockSpec((1, cout), lambda i: (0, 0)),
        ],
        out_specs=pl.BlockSpec((1, himg * wimg, cout), lambda i: (i, 0, 0)),
        out_shape=jax.ShapeDtypeStruct((b, himg * wimg, cout), jnp.float32),
        compiler_params=pltpu.CompilerParams(dimension_semantics=("parallel",)),
    )(xp, w9, s, t)


def _dcn_body(g_ref, w_ref, s_ref, t_ref, o_ref):
    blk, c = g_ref.shape[2], g_ref.shape[3]
    acc = jnp.zeros((blk, w_ref.shape[2]), jnp.float32)
    for k in range(9):
        acc = acc + jnp.dot(g_ref[0, k], w_ref[k],
                            preferred_element_type=jnp.float32)
    o_ref[0] = jnp.maximum(acc * s_ref[...] + t_ref[...], 0.0)


def _dcn_contract(g, w9, s, t, blk=4096):
    b, _, hw, c = g.shape
    return pl.pallas_call(
        _dcn_body,
        grid=(b, hw // blk),
        in_specs=[
            pl.BlockSpec((1, 9, blk, c), lambda i, j: (i, 0, j, 0)),
            pl.BlockSpec((9, c, c), lambda i, j: (0, 0, 0)),
            pl.BlockSpec((1, c), lambda i, j: (0, 0)),
            pl.BlockSpec((1, c), lambda i, j: (0, 0)),
        ],
        out_specs=pl.BlockSpec((1, blk, c), lambda i, j: (i, j, 0)),
        out_shape=jax.ShapeDtypeStruct((b, hw, c), jnp.float32),
        compiler_params=pltpu.CompilerParams(
            dimension_semantics=("parallel", "arbitrary")),
    )(g, w9, s, t)


def _conv9_split(xp, w9, s, t, act, dil):
    himg = xp.shape[1] - 2 * dil
    part = himg // 4
    outs = [_conv9(xp[:, p * part:(p + 1) * part + 2 * dil], w9, s, t, act, dil)
            for p in range(4)]
    return jnp.concatenate(outs, axis=1)


def _tap_w(w):
    return w.transpose(2, 3, 1, 0).reshape(9, w.shape[1], w.shape[0])


def kernel(x, w_om, b_om, w_dcn, b_dcn,
           bn1_g, bn1_b, bn1_m, bn1_v,
           w_h, b_h, bn2_g, bn2_b, bn2_m, bn2_v,
           w_w, b_w, bn3_g, bn3_b, bn3_m, bn3_v,
           w3, b3):
    b, c, h, w = x.shape
    hw = h * w
    f32 = jnp.float32

    xh = x.transpose(0, 2, 3, 1)
    xp1 = jnp.pad(xh, ((0, 0), (1, 1), (1, 1), (0, 0)))

    om = _conv9_split(xp1, _tap_w(w_om), jnp.ones((1, 27), f32),
                b_om.reshape(1, 27), act=None, dil=1)

    o1 = om[:, :, 0:9].transpose(0, 2, 1)
    o2 = om[:, :, 9:18].transpose(0, 2, 1)
    mk = jax.nn.sigmoid(om[:, :, 18:27]).transpose(0, 2, 1)

    kyv = jnp.repeat(jnp.arange(3, dtype=f32) - 1, 3)
    kxv = jnp.tile(jnp.arange(3, dtype=f32) - 1, 3)
    rows = (jnp.arange(hw, dtype=jnp.int32) // w).astype(f32)
    cols = (jnp.arange(hw, dtype=jnp.int32) % w).astype(f32)
    py = rows[None, None, :] + kyv[None, :, None] + o1
    px = cols[None, None, :] + kxv[None, :, None] + o2
    y0 = jnp.floor(py)
    x0 = jnp.floor(px)
    wy = py - y0
    wx = px - x0
    y0i = y0.astype(jnp.int32)
    x0i = x0.astype(jnp.int32)

    sy = jnp.clip(y0i, -1, h - 1) + 1
    sx = jnp.clip(x0i, -1, w - 1) + 1
    starts = jnp.stack([sy.reshape(b, -1), sx.reshape(b, -1)], axis=-1)
    dnums = jax.lax.GatherDimensionNumbers(
        offset_dims=(1, 2, 3), collapsed_slice_dims=(), start_index_map=(0, 1))
    patch = jax.vmap(lambda op, st: jax.lax.gather(
        op, st, dnums, slice_sizes=(2, 2, c),
        mode=jax.lax.GatherScatterMode.CLIP))(xp1, starts)
    g = jnp.zeros((b, 9 * hw, c), f32)
    for dy in (0, 1):
        for dx in (0, 1):
            yi = y0i + dy
            xi = x0i + dx
            valid = (yi >= 0) & (yi < h) & (xi >= 0) & (xi < w)
            wc = jnp.where(valid,
                           (wy if dy else 1.0 - wy) * (wx if dx else 1.0 - wx),
                           0.0) * mk
            g = g + patch[:, :, dy, dx, :] * wc.reshape(b, -1)[..., None]
    g = g.reshape(b, 9, hw, c)

    s1 = bn1_g * jax.lax.rsqrt(bn1_v + _BN_EPS)
    t1 = (b_dcn - bn1_m) * s1 + bn1_b
    h1 = _dcn_contract(g, _tap_w(w_dcn), s1.reshape(1, c), t1.reshape(1, c))

    s2 = bn2_g * jax.lax.rsqrt(bn2_v + _BN_EPS)
    t2 = (b_h - bn2_m) * s2 + bn2_b
    hp = jnp.pad(h1.reshape(b, h, w, c), ((0, 0), (2, 2), (2, 2), (0, 0)))
    h2 = _conv9_split(hp, _tap_w(w_h), s2.reshape(1, c), t2.reshape(1, c),
                act='relu', dil=2)

    s3 = bn3_g * jax.lax.rsqrt(bn3_v + _BN_EPS)
    t3 = (b_w - bn3_m) * s3 + bn3_b
    hp = jnp.pad(h2.reshape(b, h, w, c), ((0, 0), (2, 2), (2, 2), (0, 0)))
    h3 = _conv9_split(hp, _tap_w(w_w), s3.reshape(1, c), t3.reshape(1, c),
                act='relu', dil=2)

    hp = jnp.pad(h3.reshape(b, h, w, c), ((0, 0), (1, 1), (1, 1), (0, 0)))
    out = _conv9_split(hp, _tap_w(w3), jnp.ones((1, 1), f32), b3.reshape(1, 1),
                 act='sig', dil=1)
    return out.reshape(b, h, w, 1).transpose(0, 3, 1, 2)
